# Initial kernel scaffold; baseline (speedup 1.0000x reference)
import jax
import jax.numpy as jnp
from jax.experimental import pallas as pl


def kernel(categorical_features, continuous_features, E0, E1, E2, W, b):
    raise NotImplementedError("write your pallas kernel here")


# speedup vs baseline: 3.4667x; 3.4667x over previous
import jax
import jax.numpy as jnp
from jax.experimental import pallas as pl
from jax.experimental.pallas import tpu as pltpu

_RAW_DIM = 145
_OH_DIM = 137
_OUT_DIM = 128
_CONT_DIM = 8
_BLOCK = 2000


def _fused_kernel(cat_ref, cont_ref, e0_ref, e1_ref, e2_ref, w_ref, b_ref,
                  proj_ref, raw_ref, bt_ref):
    i = pl.program_id(0)

    @pl.when(i == 0)
    def _build_table():
        t0 = jnp.dot(e0_ref[...], w_ref[0:64, :],
                     preferred_element_type=jnp.float32)
        t1 = jnp.dot(e1_ref[...], w_ref[64:80, :],
                     preferred_element_type=jnp.float32)
        t2 = jnp.dot(e2_ref[...], w_ref[80:96, :],
                     preferred_element_type=jnp.float32)
        pad = jnp.zeros((_RAW_DIM - _OH_DIM, _OUT_DIM), jnp.float32)
        bt_ref[...] = jnp.concatenate([t0, t1, t2, pad], axis=0)

    cat = cat_ref[...]
    cont = cont_ref[...]
    col = jax.lax.broadcasted_iota(jnp.int32, (_BLOCK, _RAW_DIM), 1)
    i0 = cat[:, 0:1]
    i1 = cat[:, 1:2] + 119
    i2 = cat[:, 2:3] + 129
    oh = ((col == i0) | (col == i1) | (col == i2)).astype(jnp.float32)

    raw_ref[...] = oh
    raw_ref[:, _OH_DIM:_RAW_DIM] = cont

    proj = jnp.dot(oh, bt_ref[...], preferred_element_type=jnp.float32)
    proj += jnp.dot(cont, w_ref[96:104, :], preferred_element_type=jnp.float32)
    proj_ref[...] = proj + b_ref[...]


@jax.jit
def kernel(categorical_features, continuous_features, E0, E1, E2, W, b):
    n = categorical_features.shape[0]
    cat = categorical_features.astype(jnp.int32)
    b2 = b.reshape(1, _OUT_DIM)
    grid = n // _BLOCK

    proj, raw = pl.pallas_call(
        _fused_kernel,
        grid=(grid,),
        in_specs=[
            pl.BlockSpec((_BLOCK, 3), lambda i: (i, 0)),
            pl.BlockSpec((_BLOCK, _CONT_DIM), lambda i: (i, 0)),
            pl.BlockSpec(E0.shape, lambda i: (0, 0)),
            pl.BlockSpec(E1.shape, lambda i: (0, 0)),
            pl.BlockSpec(E2.shape, lambda i: (0, 0)),
            pl.BlockSpec(W.shape, lambda i: (0, 0)),
            pl.BlockSpec((1, _OUT_DIM), lambda i: (0, 0)),
        ],
        out_specs=[
            pl.BlockSpec((_BLOCK, _OUT_DIM), lambda i: (i, 0)),
            pl.BlockSpec((_BLOCK, _RAW_DIM), lambda i: (i, 0)),
        ],
        out_shape=[
            jax.ShapeDtypeStruct((n, _OUT_DIM), jnp.float32),
            jax.ShapeDtypeStruct((n, _RAW_DIM), jnp.float32),
        ],
        scratch_shapes=[pltpu.VMEM((_RAW_DIM, _OUT_DIM), jnp.float32)],
    )(cat, continuous_features, E0, E1, E2, W, b2)
    return proj, raw


# speedup vs baseline: 13.3377x; 3.8473x over previous
import jax
import jax.numpy as jnp
from jax.experimental import pallas as pl
from jax.experimental.pallas import tpu as pltpu

_RAW_DIM = 145
_OH_DIM = 137
_OUT_DIM = 128
_CONT_DIM = 8
_BLOCK = 2048


def _fused_kernel(cat_ref, cont_ref, e0_ref, e1_ref, e2_ref, w_ref, b_ref,
                  proj_ref, rawt_ref, bt_ref):
    i = pl.program_id(0)

    @pl.when(i == 0)
    def _build_table():
        t0 = jnp.dot(e0_ref[...], w_ref[0:64, :],
                     preferred_element_type=jnp.float32)
        t1 = jnp.dot(e1_ref[...], w_ref[64:80, :],
                     preferred_element_type=jnp.float32)
        t2 = jnp.dot(e2_ref[...], w_ref[80:96, :],
                     preferred_element_type=jnp.float32)
        bt_ref[...] = jnp.concatenate([t0, t1, t2, w_ref[96:104, :]], axis=0)

    idx = cat_ref[...]
    cont = cont_ref[...]
    row = jax.lax.broadcasted_iota(jnp.int32, (_OH_DIM, _BLOCK), 0)
    i0 = idx[0:1, :]
    i1 = idx[1:2, :] + 119
    i2 = idx[2:3, :] + 129
    oh = ((row == i0) | (row == i1) | (row == i2)).astype(jnp.float32)

    full = jnp.concatenate([oh, cont], axis=0)
    rawt_ref[...] = full

    proj = jax.lax.dot_general(full, bt_ref[...],
                               (((0,), (0,)), ((), ())),
                               preferred_element_type=jnp.float32)
    proj_ref[...] = proj + b_ref[...]


@jax.jit
def kernel(categorical_features, continuous_features, E0, E1, E2, W, b):
    n = categorical_features.shape[0]
    cat_t = categorical_features.astype(jnp.int32).T
    cont_t = continuous_features.T
    b2 = b.reshape(1, _OUT_DIM)
    grid = pl.cdiv(n, _BLOCK)

    proj, raw_t = pl.pallas_call(
        _fused_kernel,
        grid=(grid,),
        in_specs=[
            pl.BlockSpec((3, _BLOCK), lambda i: (0, i)),
            pl.BlockSpec((_CONT_DIM, _BLOCK), lambda i: (0, i)),
            pl.BlockSpec(E0.shape, lambda i: (0, 0)),
            pl.BlockSpec(E1.shape, lambda i: (0, 0)),
            pl.BlockSpec(E2.shape, lambda i: (0, 0)),
            pl.BlockSpec(W.shape, lambda i: (0, 0)),
            pl.BlockSpec((1, _OUT_DIM), lambda i: (0, 0)),
        ],
        out_specs=[
            pl.BlockSpec((_BLOCK, _OUT_DIM), lambda i: (i, 0)),
            pl.BlockSpec((_RAW_DIM, _BLOCK), lambda i: (0, i)),
        ],
        out_shape=[
            jax.ShapeDtypeStruct((n, _OUT_DIM), jnp.float32),
            jax.ShapeDtypeStruct((_RAW_DIM, n), jnp.float32),
        ],
        scratch_shapes=[pltpu.VMEM((_RAW_DIM, _OUT_DIM), jnp.float32)],
    )(cat_t, cont_t, E0, E1, E2, W, b2)
    return proj, raw_t.T


# speedup vs baseline: 13.9575x; 1.0465x over previous
import jax
import jax.numpy as jnp
from jax.experimental import pallas as pl
from jax.experimental.pallas import tpu as pltpu

_RAW_DIM = 145
_OH_DIM = 137
_OUT_DIM = 128
_CONT_DIM = 8
_BLOCK = 2048


def _fused_kernel(cat_ref, cont_ref, e0_ref, e1_ref, e2_ref, w_ref, b_ref,
                  proj_ref, rawt_ref, bt_ref):
    i = pl.program_id(0)

    @pl.when(i == 0)
    def _build_table():
        t0 = jnp.dot(e0_ref[...], w_ref[0:64, :],
                     preferred_element_type=jnp.float32)
        t1 = jnp.dot(e1_ref[...], w_ref[64:80, :],
                     preferred_element_type=jnp.float32)
        t2 = jnp.dot(e2_ref[...], w_ref[80:96, :],
                     preferred_element_type=jnp.float32)
        bt = jnp.concatenate([t0, t1, t2, w_ref[96:104, :]], axis=0)
        bt_ref[...] = bt.astype(jnp.bfloat16)

    idx = cat_ref[...]
    cont = cont_ref[...]
    i0 = idx[0:1, :]
    i1 = idx[1:2, :] + 119
    i2 = idx[2:3, :] + 129
    row_a = jax.lax.broadcasted_iota(jnp.int32, (112, _BLOCK), 0)
    oh_a = (row_a == i0).astype(jnp.float32)
    row_b = jax.lax.broadcasted_iota(jnp.int32, (_OH_DIM - 112, _BLOCK), 0) + 112
    oh_b = ((row_b == i0) | (row_b == i1) | (row_b == i2)).astype(jnp.float32)

    full = jnp.concatenate([oh_a, oh_b, cont], axis=0)
    rawt_ref[...] = full

    proj = jax.lax.dot_general(full.astype(jnp.bfloat16), bt_ref[...],
                               (((0,), (0,)), ((), ())),
                               preferred_element_type=jnp.float32)
    proj_ref[...] = proj + b_ref[...]


@jax.jit
def kernel(categorical_features, continuous_features, E0, E1, E2, W, b):
    n = categorical_features.shape[0]
    cat_t = categorical_features.astype(jnp.int32).T
    cont_t = continuous_features.T
    b2 = b.reshape(1, _OUT_DIM)
    grid = pl.cdiv(n, _BLOCK)

    proj, raw_t = pl.pallas_call(
        _fused_kernel,
        grid=(grid,),
        in_specs=[
            pl.BlockSpec((3, _BLOCK), lambda i: (0, i)),
            pl.BlockSpec((_CONT_DIM, _BLOCK), lambda i: (0, i)),
            pl.BlockSpec(E0.shape, lambda i: (0, 0)),
            pl.BlockSpec(E1.shape, lambda i: (0, 0)),
            pl.BlockSpec(E2.shape, lambda i: (0, 0)),
            pl.BlockSpec(W.shape, lambda i: (0, 0)),
            pl.BlockSpec((1, _OUT_DIM), lambda i: (0, 0)),
        ],
        out_specs=[
            pl.BlockSpec((_BLOCK, _OUT_DIM), lambda i: (i, 0)),
            pl.BlockSpec((_RAW_DIM, _BLOCK), lambda i: (0, i)),
        ],
        out_shape=[
            jax.ShapeDtypeStruct((n, _OUT_DIM), jnp.float32),
            jax.ShapeDtypeStruct((_RAW_DIM, n), jnp.float32),
        ],
        scratch_shapes=[pltpu.VMEM((_RAW_DIM, _OUT_DIM), jnp.bfloat16)],
    )(cat_t, cont_t, E0, E1, E2, W, b2)
    return proj, raw_t.T


# speedup vs baseline: 18.1730x; 1.3020x over previous
import jax
import jax.numpy as jnp
from jax.experimental import pallas as pl
from jax.experimental.pallas import tpu as pltpu

_RAW_DIM = 145
_OH_DIM = 137
_OUT_DIM = 128
_CONT_DIM = 8
_BLOCK = 4096


def _fused_kernel(cat_ref, cont_ref, e0_ref, e1_ref, e2_ref, w_ref, b_ref,
                  proj_ref, rawt_ref, bt_ref):
    i = pl.program_id(0)

    @pl.when(i == 0)
    def _build_table():
        t0 = jnp.dot(e0_ref[...], w_ref[0:64, :],
                     preferred_element_type=jnp.float32)
        t1 = jnp.dot(e1_ref[...], w_ref[64:80, :],
                     preferred_element_type=jnp.float32)
        t2 = jnp.dot(e2_ref[...], w_ref[80:96, :],
                     preferred_element_type=jnp.float32)
        bt = jnp.concatenate([t0, t1, t2, w_ref[96:104, :]], axis=0)
        bt_ref[...] = bt.astype(jnp.bfloat16)

    idx = cat_ref[...]
    cont = cont_ref[...]
    i0 = idx[0:1, :]
    i1 = idx[1:2, :] + 119
    i2 = idx[2:3, :] + 129
    row_a = jax.lax.broadcasted_iota(jnp.int32, (112, _BLOCK), 0)
    oh_a = (row_a == i0).astype(jnp.float32)
    row_b = jax.lax.broadcasted_iota(jnp.int32, (_OH_DIM - 112, _BLOCK), 0) + 112
    oh_b = ((row_b == i0) | (row_b == i1) | (row_b == i2)).astype(jnp.float32)

    full = jnp.concatenate([oh_a, oh_b, cont], axis=0)
    rawt_ref[...] = full

    proj = jax.lax.dot_general(full.astype(jnp.bfloat16), bt_ref[...],
                               (((0,), (0,)), ((), ())),
                               preferred_element_type=jnp.float32)
    proj_ref[...] = proj + b_ref[...]


@jax.jit
def kernel(categorical_features, continuous_features, E0, E1, E2, W, b):
    n = categorical_features.shape[0]
    cat_t = categorical_features.astype(jnp.int32).T
    cont_t = continuous_features.T
    b2 = b.reshape(1, _OUT_DIM)
    grid = pl.cdiv(n, _BLOCK)

    proj, raw_t = pl.pallas_call(
        _fused_kernel,
        grid=(grid,),
        in_specs=[
            pl.BlockSpec((3, _BLOCK), lambda i: (0, i)),
            pl.BlockSpec((_CONT_DIM, _BLOCK), lambda i: (0, i)),
            pl.BlockSpec(E0.shape, lambda i: (0, 0)),
            pl.BlockSpec(E1.shape, lambda i: (0, 0)),
            pl.BlockSpec(E2.shape, lambda i: (0, 0)),
            pl.BlockSpec(W.shape, lambda i: (0, 0)),
            pl.BlockSpec((1, _OUT_DIM), lambda i: (0, 0)),
        ],
        out_specs=[
            pl.BlockSpec((_BLOCK, _OUT_DIM), lambda i: (i, 0)),
            pl.BlockSpec((_RAW_DIM, _BLOCK), lambda i: (0, i)),
        ],
        out_shape=[
            jax.ShapeDtypeStruct((n, _OUT_DIM), jnp.float32),
            jax.ShapeDtypeStruct((_RAW_DIM, n), jnp.float32),
        ],
        scratch_shapes=[pltpu.VMEM((_RAW_DIM, _OUT_DIM), jnp.bfloat16)],
    )(cat_t, cont_t, E0, E1, E2, W, b2)
    return proj, raw_t.T


# speedup vs baseline: 20.3369x; 1.1191x over previous
import jax
import jax.numpy as jnp
from jax.experimental import pallas as pl
from jax.experimental.pallas import tpu as pltpu

_RAW_DIM = 145
_OH_DIM = 137
_OUT_DIM = 128
_CONT_DIM = 8
_BLOCK = 8192


def _fused_kernel(cat_ref, cont_ref, e0_ref, e1_ref, e2_ref, w_ref, b_ref,
                  proj_ref, rawt_ref, bt_ref):
    i = pl.program_id(0)

    @pl.when(i == 0)
    def _build_table():
        t0 = jnp.dot(e0_ref[...], w_ref[0:64, :],
                     preferred_element_type=jnp.float32)
        t1 = jnp.dot(e1_ref[...], w_ref[64:80, :],
                     preferred_element_type=jnp.float32)
        t2 = jnp.dot(e2_ref[...], w_ref[80:96, :],
                     preferred_element_type=jnp.float32)
        bt = jnp.concatenate([t0, t1, t2, w_ref[96:104, :]], axis=0)
        bt_ref[...] = bt.astype(jnp.bfloat16)

    idx = cat_ref[...]
    cont = cont_ref[...]
    i0 = idx[0:1, :]
    i1 = idx[1:2, :] + 119
    i2 = idx[2:3, :] + 129
    row_a = jax.lax.broadcasted_iota(jnp.int32, (112, _BLOCK), 0)
    oh_a = (row_a == i0).astype(jnp.float32)
    row_b = jax.lax.broadcasted_iota(jnp.int32, (_OH_DIM - 112, _BLOCK), 0) + 112
    oh_b = ((row_b == i0) | (row_b == i1) | (row_b == i2)).astype(jnp.float32)

    full = jnp.concatenate([oh_a, oh_b, cont], axis=0)
    rawt_ref[...] = full

    proj = jax.lax.dot_general(full.astype(jnp.bfloat16), bt_ref[...],
                               (((0,), (0,)), ((), ())),
                               preferred_element_type=jnp.float32)
    proj_ref[...] = proj + b_ref[...]


@jax.jit
def kernel(categorical_features, continuous_features, E0, E1, E2, W, b):
    n = categorical_features.shape[0]
    cat_t = categorical_features.astype(jnp.int32).T
    cont_t = continuous_features.T
    b2 = b.reshape(1, _OUT_DIM)
    grid = pl.cdiv(n, _BLOCK)

    proj, raw_t = pl.pallas_call(
        _fused_kernel,
        grid=(grid,),
        in_specs=[
            pl.BlockSpec((3, _BLOCK), lambda i: (0, i)),
            pl.BlockSpec((_CONT_DIM, _BLOCK), lambda i: (0, i)),
            pl.BlockSpec(E0.shape, lambda i: (0, 0)),
            pl.BlockSpec(E1.shape, lambda i: (0, 0)),
            pl.BlockSpec(E2.shape, lambda i: (0, 0)),
            pl.BlockSpec(W.shape, lambda i: (0, 0)),
            pl.BlockSpec((1, _OUT_DIM), lambda i: (0, 0)),
        ],
        out_specs=[
            pl.BlockSpec((_BLOCK, _OUT_DIM), lambda i: (i, 0)),
            pl.BlockSpec((_RAW_DIM, _BLOCK), lambda i: (0, i)),
        ],
        out_shape=[
            jax.ShapeDtypeStruct((n, _OUT_DIM), jnp.float32),
            jax.ShapeDtypeStruct((_RAW_DIM, n), jnp.float32),
        ],
        scratch_shapes=[pltpu.VMEM((_RAW_DIM, _OUT_DIM), jnp.bfloat16)],
    )(cat_t, cont_t, E0, E1, E2, W, b2)
    return proj, raw_t.T
